# baseline (speedup 1.0000x reference)
import functools

import jax
import jax.numpy as jnp
from jax import lax
from jax.experimental import pallas as pl
from jax.experimental.pallas import tpu as pltpu
from jax.experimental.pallas import tpu_sc as plsc

_SEQ = 128
_D = 64
_C = 64
_BS = 256
_BQ = 256

_NC = 2
_NS = 16
_NW = _NC * _NS
_SC_ROWS = 1024
_RPW = _SC_ROWS // _NW


def _bcast_sum16(v):
    iota = lax.broadcasted_iota(jnp.int32, (16,), 0)
    dnums = lax.GatherDimensionNumbers(
        offset_dims=(), collapsed_slice_dims=(0,), start_index_map=(0,))
    for sh in (8, 4, 2, 1):
        idx = jnp.bitwise_xor(iota, sh)
        shuf = lax.gather(v, idx[:, None], dnums, (1,),
                          mode=lax.GatherScatterMode.PROMISE_IN_BOUNDS)
        v = v + shuf
    return v


def _sc_body(n_sup, sup_hbm, labels_hbm, w_hbm, parts_hbm,
             x_v, lbl_v, w_v, acc_v, sem):
    wid = lax.axis_index("s") * _NC + lax.axis_index("c")
    base = (n_sup - _SC_ROWS) + wid * _RPW
    pltpu.sync_copy(w_hbm, w_v)
    pltpu.sync_copy(labels_hbm.at[pl.ds(base, _RPW)], lbl_v)

    zero16 = jnp.zeros((16,), jnp.float32)

    def zero_body(i, _):
        acc_v[i, pl.ds(0, 16)] = zero16
        acc_v[i, pl.ds(16, 16)] = zero16
        acc_v[i, pl.ds(32, 16)] = zero16
        acc_v[i, pl.ds(48, 16)] = zero16
        return 0

    lax.fori_loop(0, _C, zero_body, 0)

    def chunk_body(k, _):
        lbl16 = lbl_v[pl.ds(k * 16, 16)]
        for j in range(16):
            r = k * 16 + j
            pltpu.sync_copy(sup_hbm.at[base + r], x_v)

            def d_body(d, carry):
                s8 = x_v[d, pl.ds(0, 16)]
                for c in range(1, 8):
                    s8 = s8 + x_v[d, pl.ds(c * 16, 16)]
                s = _bcast_sum16(s8)
                return tuple(
                    carry[c] + s * w_v[d, pl.ds(c * 16, 16)]
                    for c in range(4)
                )

            emb = lax.fori_loop(0, _D, d_body,
                                (zero16, zero16, zero16, zero16))
            lbl = lbl16[j]
            for c in range(4):
                plsc.addupdate(acc_v.at[lbl, pl.ds(c * 16, 16)], emb[c])
        return 0

    lax.fori_loop(0, _RPW // 16, chunk_body, 0)
    pltpu.sync_copy(acc_v, parts_hbm.at[wid])


def sc_partial_sums(sup_t, labels_i32, W):
    n_sup = sup_t.shape[0]
    mesh = plsc.VectorSubcoreMesh(core_axis_name="c", subcore_axis_name="s")
    w_scaled = W * (1.0 / _SEQ)
    return pl.kernel(
        functools.partial(_sc_body, n_sup),
        out_type=jax.ShapeDtypeStruct((_NW, _C, _D), jnp.float32),
        mesh=mesh,
        scratch_types=[
            pltpu.VMEM((_D, _SEQ), jnp.float32),
            pltpu.VMEM((_RPW,), jnp.int32),
            pltpu.VMEM((_D, _D), jnp.float32),
            pltpu.VMEM((_C, _D), jnp.float32),
            pltpu.SemaphoreType.DMA,
        ],
    )(sup_t, labels_i32, w_scaled)


def _pool_project(x, w):
    pooled = jnp.sum(x, axis=2) * (1.0 / _SEQ)
    return jnp.dot(pooled, w, preferred_element_type=jnp.float32)


def _support_body(nbs_tc, labels_ref, x_ref, w_ref, sums_ref, counts_ref):
    i = pl.program_id(0)
    lbl = labels_ref[0, 0, :]
    onehot = (lbl[:, None] == lax.broadcasted_iota(jnp.int32, (_BS, _C), 1)
              ).astype(jnp.float32)
    ones_col = jnp.ones((_BS, 1), jnp.float32)
    part_counts = lax.dot_general(onehot, ones_col, (((0,), (0,)), ((), ())),
                                  preferred_element_type=jnp.float32)

    @pl.when(i == 0)
    def _():
        sums_ref[...] = jnp.zeros((_C, _D), jnp.float32)
        counts_ref[...] = part_counts

    @pl.when(i > 0)
    def _():
        counts_ref[...] += part_counts

    @pl.when(i < nbs_tc)
    def _():
        emb = _pool_project(x_ref[...], w_ref[...])
        part_sums = lax.dot_general(onehot, emb, (((0,), (0,)), ((), ())),
                                    preferred_element_type=jnp.float32)
        sums_ref[...] += part_sums


def _query_body(x_ref, w_ref, b_ref, sums_ref, counts_ref, parts_ref,
                logits_t_ref, protos_ref):
    j = pl.program_id(0)
    counts = counts_ref[...]
    denom = jnp.maximum(counts, 1.0)
    sums = sums_ref[...] + jnp.sum(parts_ref[...], axis=0)
    protos = (sums + counts * b_ref[...]) / denom

    @pl.when(j == 0)
    def _():
        protos_ref[...] = protos

    qe = _pool_project(x_ref[...], w_ref[...]) + b_ref[...]
    p2 = jnp.sum(protos * protos, axis=1, keepdims=True)
    ones_row = jnp.ones((1, _D), jnp.float32)
    q2t = lax.dot_general(ones_row, qe * qe, (((1,), (1,)), ((), ())),
                          preferred_element_type=jnp.float32)
    cross_t = lax.dot_general(protos, qe, (((1,), (1,)), ((), ())),
                              preferred_element_type=jnp.float32)
    logits_t_ref[...] = -(p2 + q2t - 2.0 * cross_t + 1e-8)


@jax.jit
def kernel(support, support_labels, query, W, b):
    n_sup = support.shape[0]
    n_q = query.shape[0]
    nbs = n_sup // _BS
    nbs_tc = (n_sup - _SC_ROWS) // _BS
    nbq = n_q // _BQ
    sup_t = support.transpose(0, 2, 1)
    q_t = query.transpose(0, 2, 1)
    labels_i32 = support_labels.astype(jnp.int32)
    labels = labels_i32.reshape(nbs, 1, _BS)
    b_row = b.reshape(1, _D)

    parts = sc_partial_sums(sup_t, labels_i32, W)

    sums, counts = pl.pallas_call(
        functools.partial(_support_body, nbs_tc),
        grid=(nbs,),
        in_specs=[
            pl.BlockSpec((1, 1, _BS), lambda i: (i, 0, 0)),
            pl.BlockSpec((_BS, _D, _SEQ),
                         lambda i: (jnp.minimum(i, nbs_tc - 1), 0, 0)),
            pl.BlockSpec((_D, _D), lambda i: (0, 0)),
        ],
        out_specs=[
            pl.BlockSpec((_C, _D), lambda i: (0, 0)),
            pl.BlockSpec((_C, 1), lambda i: (0, 0)),
        ],
        out_shape=[
            jax.ShapeDtypeStruct((_C, _D), jnp.float32),
            jax.ShapeDtypeStruct((_C, 1), jnp.float32),
        ],
    )(labels, sup_t, W)

    logits_t, protos = pl.pallas_call(
        _query_body,
        grid=(nbq,),
        in_specs=[
            pl.BlockSpec((_BQ, _D, _SEQ), lambda j: (j, 0, 0)),
            pl.BlockSpec((_D, _D), lambda j: (0, 0)),
            pl.BlockSpec((1, _D), lambda j: (0, 0)),
            pl.BlockSpec((_C, _D), lambda j: (0, 0)),
            pl.BlockSpec((_C, 1), lambda j: (0, 0)),
            pl.BlockSpec((_NW, _C, _D), lambda j: (0, 0, 0)),
        ],
        out_specs=[
            pl.BlockSpec((_C, _BQ), lambda j: (0, j)),
            pl.BlockSpec((_C, _D), lambda j: (0, 0)),
        ],
        out_shape=[
            jax.ShapeDtypeStruct((_C, n_q), jnp.float32),
            jax.ShapeDtypeStruct((_C, _D), jnp.float32),
        ],
    )(q_t, W, b_row, sums, counts, parts)

    return (logits_t.T, protos)


# speedup vs baseline: 1.3937x; 1.3937x over previous
import jax
import jax.numpy as jnp
from jax import lax
from jax.experimental import pallas as pl

_SEQ = 128
_D = 64
_C = 64
_BS = 256
_BQ = 256


def _pool_project(x, w):
    pooled = jnp.sum(x, axis=2) * (1.0 / _SEQ)
    return jnp.dot(pooled, w, preferred_element_type=jnp.float32)


def _support_body(labels_ref, x_ref, w_ref, sums_ref, counts_ref):
    i = pl.program_id(0)
    emb = _pool_project(x_ref[...], w_ref[...])
    lbl = labels_ref[0, 0, :]
    onehot = (lbl[:, None] == lax.broadcasted_iota(jnp.int32, (_BS, _C), 1)
              ).astype(jnp.float32)
    part_sums = lax.dot_general(onehot, emb, (((0,), (0,)), ((), ())),
                                preferred_element_type=jnp.float32)
    ones_col = jnp.ones((_BS, 1), jnp.float32)
    part_counts = lax.dot_general(onehot, ones_col, (((0,), (0,)), ((), ())),
                                  preferred_element_type=jnp.float32)

    @pl.when(i == 0)
    def _():
        sums_ref[...] = part_sums
        counts_ref[...] = part_counts

    @pl.when(i > 0)
    def _():
        sums_ref[...] += part_sums
        counts_ref[...] += part_counts


def _query_body(x_ref, w_ref, b_ref, sums_ref, counts_ref,
                logits_t_ref, protos_ref):
    j = pl.program_id(0)
    counts = counts_ref[...]
    denom = jnp.maximum(counts, 1.0)
    protos = (sums_ref[...] + counts * b_ref[...]) / denom

    @pl.when(j == 0)
    def _():
        protos_ref[...] = protos

    qe = _pool_project(x_ref[...], w_ref[...]) + b_ref[...]
    p2 = jnp.sum(protos * protos, axis=1, keepdims=True)
    ones_row = jnp.ones((1, _D), jnp.float32)
    q2t = lax.dot_general(ones_row, qe * qe, (((1,), (1,)), ((), ())),
                          preferred_element_type=jnp.float32)
    cross_t = lax.dot_general(protos, qe, (((1,), (1,)), ((), ())),
                              preferred_element_type=jnp.float32)
    logits_t_ref[...] = -(p2 + q2t - 2.0 * cross_t + 1e-8)


@jax.jit
def kernel(support, support_labels, query, W, b):
    n_sup = support.shape[0]
    n_q = query.shape[0]
    nbs = n_sup // _BS
    nbq = n_q // _BQ
    sup_t = support.transpose(0, 2, 1)
    q_t = query.transpose(0, 2, 1)
    labels = support_labels.astype(jnp.int32).reshape(nbs, 1, _BS)
    b_row = b.reshape(1, _D)

    sums, counts = pl.pallas_call(
        _support_body,
        grid=(nbs,),
        in_specs=[
            pl.BlockSpec((1, 1, _BS), lambda i: (i, 0, 0)),
            pl.BlockSpec((_BS, _D, _SEQ), lambda i: (i, 0, 0)),
            pl.BlockSpec((_D, _D), lambda i: (0, 0)),
        ],
        out_specs=[
            pl.BlockSpec((_C, _D), lambda i: (0, 0)),
            pl.BlockSpec((_C, 1), lambda i: (0, 0)),
        ],
        out_shape=[
            jax.ShapeDtypeStruct((_C, _D), jnp.float32),
            jax.ShapeDtypeStruct((_C, 1), jnp.float32),
        ],
    )(labels, sup_t, W)

    logits_t, protos = pl.pallas_call(
        _query_body,
        grid=(nbq,),
        in_specs=[
            pl.BlockSpec((_BQ, _D, _SEQ), lambda j: (j, 0, 0)),
            pl.BlockSpec((_D, _D), lambda j: (0, 0)),
            pl.BlockSpec((1, _D), lambda j: (0, 0)),
            pl.BlockSpec((_C, _D), lambda j: (0, 0)),
            pl.BlockSpec((_C, 1), lambda j: (0, 0)),
        ],
        out_specs=[
            pl.BlockSpec((_C, _BQ), lambda j: (0, j)),
            pl.BlockSpec((_C, _D), lambda j: (0, 0)),
        ],
        out_shape=[
            jax.ShapeDtypeStruct((_C, n_q), jnp.float32),
            jax.ShapeDtypeStruct((_C, _D), jnp.float32),
        ],
    )(q_t, W, b_row, sums, counts)

    return (logits_t.T, protos)


# speedup vs baseline: 1.4800x; 1.0620x over previous
import jax
import jax.numpy as jnp
from jax import lax
from jax.experimental import pallas as pl
from jax.experimental.pallas import tpu as pltpu

_SEQ = 128
_D = 64
_C = 64
_BS = 512
_BQ = 512


def _pool_project(x, w):
    pooled = jnp.sum(x, axis=2) * (1.0 / _SEQ)
    return jnp.dot(pooled, w, preferred_element_type=jnp.float32)


def _support_body(labels_ref, x_ref, w_ref, sums_ref, counts_ref):
    i = pl.program_id(0)
    emb = _pool_project(x_ref[...], w_ref[...])
    lbl = labels_ref[0, 0, :]
    onehot = (lbl[:, None] == lax.broadcasted_iota(jnp.int32, (_BS, _C), 1)
              ).astype(jnp.float32)
    part_sums = lax.dot_general(onehot, emb, (((0,), (0,)), ((), ())),
                                preferred_element_type=jnp.float32)
    ones_col = jnp.ones((_BS, 1), jnp.float32)
    part_counts = lax.dot_general(onehot, ones_col, (((0,), (0,)), ((), ())),
                                  preferred_element_type=jnp.float32)

    @pl.when(i == 0)
    def _():
        sums_ref[...] = part_sums
        counts_ref[...] = part_counts

    @pl.when(i > 0)
    def _():
        sums_ref[...] += part_sums
        counts_ref[...] += part_counts


def _query_body(x_ref, w_ref, b_ref, sums_ref, counts_ref,
                logits_t_ref, protos_ref):
    j = pl.program_id(0)
    counts = counts_ref[...]
    denom = jnp.maximum(counts, 1.0)
    protos = (sums_ref[...] + counts * b_ref[...]) / denom

    @pl.when(j == 0)
    def _():
        protos_ref[...] = protos

    qe = _pool_project(x_ref[...], w_ref[...]) + b_ref[...]
    p2 = jnp.sum(protos * protos, axis=1, keepdims=True)
    ones_row = jnp.ones((1, _D), jnp.float32)
    q2t = lax.dot_general(ones_row, qe * qe, (((1,), (1,)), ((), ())),
                          preferred_element_type=jnp.float32)
    cross_t = lax.dot_general(protos, qe, (((1,), (1,)), ((), ())),
                              preferred_element_type=jnp.float32)
    logits_t_ref[...] = -(p2 + q2t - 2.0 * cross_t + 1e-8)


@jax.jit
def kernel(support, support_labels, query, W, b):
    n_sup = support.shape[0]
    n_q = query.shape[0]
    nbs = n_sup // _BS
    nbq = n_q // _BQ
    sup_t = support.transpose(0, 2, 1)
    q_t = query.transpose(0, 2, 1)
    labels = support_labels.astype(jnp.int32).reshape(nbs, 1, _BS)
    b_row = b.reshape(1, _D)

    sums, counts = pl.pallas_call(
        _support_body,
        grid=(nbs,),
        in_specs=[
            pl.BlockSpec((1, 1, _BS), lambda i: (i, 0, 0)),
            pl.BlockSpec((_BS, _D, _SEQ), lambda i: (i, 0, 0)),
            pl.BlockSpec((_D, _D), lambda i: (0, 0)),
        ],
        out_specs=[
            pl.BlockSpec((_C, _D), lambda i: (0, 0)),
            pl.BlockSpec((_C, 1), lambda i: (0, 0)),
        ],
        out_shape=[
            jax.ShapeDtypeStruct((_C, _D), jnp.float32),
            jax.ShapeDtypeStruct((_C, 1), jnp.float32),
        ],
        compiler_params=pltpu.CompilerParams(
            vmem_limit_bytes=100 * 1024 * 1024),
    )(labels, sup_t, W)

    logits_t, protos = pl.pallas_call(
        _query_body,
        grid=(nbq,),
        in_specs=[
            pl.BlockSpec((_BQ, _D, _SEQ), lambda j: (j, 0, 0)),
            pl.BlockSpec((_D, _D), lambda j: (0, 0)),
            pl.BlockSpec((1, _D), lambda j: (0, 0)),
            pl.BlockSpec((_C, _D), lambda j: (0, 0)),
            pl.BlockSpec((_C, 1), lambda j: (0, 0)),
        ],
        out_specs=[
            pl.BlockSpec((_C, _BQ), lambda j: (0, j)),
            pl.BlockSpec((_C, _D), lambda j: (0, 0)),
        ],
        out_shape=[
            jax.ShapeDtypeStruct((_C, n_q), jnp.float32),
            jax.ShapeDtypeStruct((_C, _D), jnp.float32),
        ],
        compiler_params=pltpu.CompilerParams(
            vmem_limit_bytes=100 * 1024 * 1024),
    )(q_t, W, b_row, sums, counts)

    return (logits_t.T, protos)
